# baseline (speedup 1.0000x reference)
import functools

import jax
import jax.numpy as jnp
from jax import lax
from jax.experimental import pallas as pl
from jax.experimental.pallas import tpu as pltpu
from jax.experimental.pallas import tpu_sc as plsc

NC = 2
NS = 16
L = 16
NW = NC * NS

B = 16384
D = 64
W = 128
FSC = 48
FTC = D - FSC
BPW = B // NW
GRP = BPW // L

_mesh = plsc.VectorSubcoreMesh(core_axis_name="c", subcore_axis_name="s")


@functools.partial(
    pl.kernel,
    out_type=jax.ShapeDtypeStruct((B,), jnp.float32),
    mesh=_mesh,
    scratch_types=[
        pltpu.VMEM((BPW,), jnp.int32),
        pltpu.VMEM((BPW,), jnp.int32),
        [pltpu.VMEM((2 * FSC, W), jnp.float32) for _ in range(2)],
        [pltpu.VMEM((2 * FSC, W), jnp.float32) for _ in range(2)],
        pltpu.VMEM((BPW,), jnp.float32),
        [pltpu.SemaphoreType.DMA for _ in range(2)],
    ],
    compiler_params=pltpu.CompilerParams(needs_layout_passes=False),
)
def _svd_sc(users_hbm, items_hbm, uemb_hbm, iemb_hbm, out_hbm,
            uidx, iidx, ustg, istg, out_v, sems):
    wid = lax.axis_index("s") * NC + lax.axis_index("c")
    base = wid * BPW

    pltpu.sync_copy(users_hbm.at[pl.ds(base, BPW)], uidx)
    pltpu.sync_copy(items_hbm.at[pl.ds(base, BPW)], iidx)

    lane = lax.iota(jnp.int32, L)

    def load_vecs(g):
        g = g & (GRP - 1)
        uv = uidx[pl.ds(g * L, L)]
        iv = iidx[pl.ds(g * L, L)]
        return (uv >> 7) << 7, uv & (W - 1), (iv >> 7) << 7, iv & (W - 1)

    def fire(vecs, e8, par):
        ucol, _, icol, _ = vecs
        for half in range(2):
            e = 2 * e8 + half
            pltpu.async_copy(
                uemb_hbm.at[pl.ds(0, FSC), pl.ds(pl.multiple_of(ucol[e], W), W)],
                ustg[par].at[pl.ds(half * FSC, FSC)], sems[par])
            pltpu.async_copy(
                iemb_hbm.at[pl.ds(0, FSC), pl.ds(pl.multiple_of(icol[e], W), W)],
                istg[par].at[pl.ds(half * FSC, FSC)], sems[par])

    def drain(par):
        dummy = uemb_hbm.at[pl.ds(0, FSC), pl.ds(0, W)]
        for buf in (ustg[par], istg[par]):
            pltpu.make_async_copy(dummy, buf.at[pl.ds(0, FSC)], sems[par]).wait()
            pltpu.make_async_copy(dummy, buf.at[pl.ds(FSC, FSC)], sems[par]).wait()

    def extract(vecs, e8, par, merged):
        _, uloc, _, iloc = vecs
        for half in range(2):
            e = 2 * e8 + half
            ucols = jnp.full((L,), 0, jnp.int32) + uloc[e]
            icols = jnp.full((L,), 0, jnp.int32) + iloc[e]
            acc = None
            for k in range(FSC // L):
                rows = half * FSC + k * L + lane
                p = plsc.load_gather(ustg[par], [rows, ucols]) * \
                    plsc.load_gather(istg[par], [rows, icols])
                acc = p if acc is None else acc + p
            merged = jnp.where(lane == e, jnp.sum(acc), merged)
        return merged

    vecs0 = load_vecs(0)
    fire(vecs0, 0, 0)

    def group_body(g, vecs):
        nvecs = load_vecs(g + 1)
        merged = jnp.zeros((L,), jnp.float32)
        for e8 in range(8):
            if e8 < 7:
                fire(vecs, e8 + 1, (e8 + 1) & 1)
            else:
                @pl.when(g < GRP - 1)
                def _(nvecs=nvecs):
                    fire(nvecs, 0, 0)
            drain(e8 & 1)
            merged = extract(vecs, e8, e8 & 1, merged)
        out_v[pl.ds(g * L, L)] = merged
        return nvecs

    lax.fori_loop(0, GRP, group_body, vecs0)

    pltpu.sync_copy(out_v, out_hbm.at[pl.ds(base, BPW)])


CHT = 128
NSTEP = B // CHT


def _tc_body(users_smem, items_smem, uemb, iemb, out_blk, ustg, istg, sems):
    i = pl.program_id(0)

    def fire(step, par):
        base = step * CHT
        for e in range(CHT):
            cu = pl.multiple_of((users_smem[base + e] >> 7) << 7, W)
            ci = pl.multiple_of((items_smem[base + e] >> 7) << 7, W)
            pltpu.make_async_copy(
                uemb.at[pl.ds(FSC, FTC), pl.ds(cu, W)],
                ustg.at[par, pl.ds(e * FTC, FTC)], sems.at[par]).start()
            pltpu.make_async_copy(
                iemb.at[pl.ds(FSC, FTC), pl.ds(ci, W)],
                istg.at[par, pl.ds(e * FTC, FTC)], sems.at[par]).start()

    def drain(par):
        dummy = uemb.at[pl.ds(FSC, FTC), pl.ds(0, W)]
        for e in range(CHT):
            pltpu.make_async_copy(
                dummy, ustg.at[par, pl.ds(e * FTC, FTC)], sems.at[par]).wait()
            pltpu.make_async_copy(
                dummy, istg.at[par, pl.ds(e * FTC, FTC)], sems.at[par]).wait()

    par = lax.rem(i, 2)
    nxt = lax.rem(i + 1, 2)

    @pl.when(i == 0)
    def _():
        fire(0, 0)

    @pl.when(i + 1 < NSTEP)
    def _():
        fire(i + 1, nxt)

    drain(par)

    base = i * CHT
    liota = jax.lax.broadcasted_iota(jnp.int32, (FTC, W), 1)
    eiota = jax.lax.broadcasted_iota(jnp.int32, (CHT,), 0)
    res = jnp.zeros((CHT,), jnp.float32)
    for e in range(CHT):
        ru = users_smem[base + e] & (W - 1)
        ri = items_smem[base + e] & (W - 1)
        uwin = ustg[par, pl.ds(e * FTC, FTC), :]
        iwin = istg[par, pl.ds(e * FTC, FTC), :]
        uv = jnp.sum(jnp.where(liota == ru, uwin, 0.0), axis=1)
        iv = jnp.sum(jnp.where(liota == ri, iwin, 0.0), axis=1)
        res = jnp.where(eiota == e, jnp.sum(uv * iv), res)
    out_blk[...] = res


_tc_call = pl.pallas_call(
    _tc_body,
    grid=(NSTEP,),
    in_specs=[
        pl.BlockSpec(memory_space=pltpu.SMEM),
        pl.BlockSpec(memory_space=pltpu.SMEM),
        pl.BlockSpec(memory_space=pl.ANY),
        pl.BlockSpec(memory_space=pl.ANY),
    ],
    out_specs=pl.BlockSpec((CHT,), lambda i: (i,)),
    out_shape=jax.ShapeDtypeStruct((B,), jnp.float32),
    scratch_shapes=[
        pltpu.VMEM((2, CHT * FTC, W), jnp.float32),
        pltpu.VMEM((2, CHT * FTC, W), jnp.float32),
        pltpu.SemaphoreType.DMA((2,)),
    ],
)


def kernel(users, items, user_emb, item_emb):
    ut, it = user_emb.T, item_emb.T
    return _svd_sc(users, items, ut, it) + _tc_call(users, items, ut, it)
